# Initial kernel scaffold; baseline (speedup 1.0000x reference)
import jax
import jax.numpy as jnp
from jax.experimental import pallas as pl


def kernel(z, pos, edge_index, batch, params):
    raise NotImplementedError("write your pallas kernel here")


# speedup vs baseline: 1.7382x; 1.7382x over previous
import functools

import jax
import jax.numpy as jnp
from jax import lax
from jax.experimental import pallas as pl
from jax.experimental.pallas import tpu as pltpu
from jax.experimental.pallas import tpu_sc as plsc

N = 10000
E = 160000
H = 128
NGAUSS = 150
NGP = 160
NI = 3
NGRAPHS = 512
NEMB = 100
CUTOFF = 15.0
LOG2 = 0.6931471805599453

NW = 32
CH = 128
E_PAD = 163840
NS = 10240

_STOP = CUTOFF - CUTOFF / NGAUSS
_STEP = _STOP / (NGAUSS - 1)
_COEFF = -0.5 / (_STEP * _STEP)


def _ssp(x):
    return jnp.maximum(x, 0.0) + jnp.log1p(jnp.exp(-jnp.abs(x))) - LOG2


def _make_gather(D, B):
    per_w = B // NW
    n_ch = per_w // CH
    mesh = plsc.VectorSubcoreMesh(core_axis_name="c", subcore_axis_name="s")

    @functools.partial(
        pl.kernel,
        out_type=jax.ShapeDtypeStruct((B, D), jnp.float32),
        mesh=mesh,
        compiler_params=pltpu.CompilerParams(use_tc_tiling_on_sc=(D % 128 == 0)),
        scratch_types=[
            pltpu.VMEM((n_ch, CH), jnp.int32),
            pltpu.VMEM((CH, D), jnp.float32),
            pltpu.SemaphoreType.DMA,
        ],
    )
    def gather_k(table_hbm, idx_hbm, out_hbm, idx_v, buf, sem):
        wid = lax.axis_index("s") * 2 + lax.axis_index("c")
        base = wid * per_w
        pltpu.sync_copy(idx_hbm.at[pl.ds(wid * n_ch, n_ch)], idx_v)

        def body(j, carry):
            pltpu.async_copy(table_hbm.at[idx_v.at[j]], buf, sem).wait()
            pltpu.sync_copy(buf, out_hbm.at[pl.ds(base + j * CH, CH)])
            return carry

        lax.fori_loop(0, n_ch, body, 0)

    return gather_k


def _make_scatter():
    n_ch = E_PAD // 2 // 16 // CH
    rows_per_tile = NS // 16
    mesh = plsc.VectorSubcoreMesh(core_axis_name="c", subcore_axis_name="s")

    @functools.partial(
        pl.kernel,
        out_type=jax.ShapeDtypeStruct((2, NS, H), jnp.float32),
        mesh=mesh,
        scratch_types=[
            pltpu.VMEM((n_ch, CH), jnp.int32),
            pltpu.VMEM((CH, H), jnp.float32),
            pltpu.VMEM_SHARED((NS, H), jnp.float32),
            pltpu.SemaphoreType.DMA,
        ],
    )
    def scatter_k(msg_hbm, idx_hbm, zeros_hbm, out_hbm, idx_v, buf, acc, sem):
        c = lax.axis_index("c")
        s = lax.axis_index("s")

        @pl.when(s == 0)
        def _():
            pltpu.sync_copy(zeros_hbm, acc)

        plsc.subcore_barrier()
        chunk0 = c * (16 * n_ch) + s * n_ch
        pltpu.sync_copy(idx_hbm.at[pl.ds(chunk0, n_ch)], idx_v)

        def body(j, carry):
            pltpu.sync_copy(msg_hbm.at[pl.ds((chunk0 + j) * CH, CH)], buf)
            pltpu.sync_copy(buf, acc.at[idx_v.at[j]], add=True)
            return carry

        lax.fori_loop(0, n_ch, body, 0)
        plsc.subcore_barrier()
        r0 = s * rows_per_tile
        pltpu.sync_copy(acc.at[pl.ds(r0, rows_per_tile)],
                        out_hbm.at[c].at[pl.ds(r0, rows_per_tile)])

    return scatter_k


_gather_h = _make_gather(H, 2 * E_PAD)
_gather_pos = _make_gather(16, 2 * E_PAD)
_scatter_msg = _make_scatter()


BE = 1024
BN = 1000
BR = 2000


def _edge_body(t0, refs):
    if t0:
        (xj_ref, xi_ref, ps_ref, pd_ref,
         w1i, w1j, w1e, b1, w2, b2, f1, fb1, f2, fb2, cf,
         enew_ref, msg_ref) = refs
        dv = pd_ref[...] - ps_ref[...]
        d2 = jnp.sum(dv * dv, axis=1, keepdims=True)
        dist = jnp.sqrt(d2)
        offs = lax.broadcasted_iota(jnp.int32, (1, NGP), 1).astype(jnp.float32) * _STEP
        e = jnp.exp(_COEFF * (dist - offs) ** 2)
    else:
        (xj_ref, xi_ref, ea_ref,
         w1i, w1j, w1e, b1, w2, b2, f1, fb1, f2, fb2, cf,
         enew_ref, msg_ref) = refs
        e = ea_ref[...]
    xj = xj_ref[...]
    xi = xi_ref[...]
    f32 = jnp.float32
    t = (jnp.dot(xi, w1i[...], preferred_element_type=f32)
         + jnp.dot(xj, w1j[...], preferred_element_type=f32)
         + jnp.dot(e, w1e[...], preferred_element_type=f32) + b1[...])
    ea = _ssp(t)
    e2 = jnp.dot(ea, w2[...], preferred_element_type=f32) + b2[...]
    enew_ref[...] = e2
    wf = _ssp(jnp.dot(e2, f1[...], preferred_element_type=f32) + fb1[...])
    wf = _ssp(jnp.dot(wf, f2[...], preferred_element_type=f32) + fb2[...])
    msg_ref[...] = jnp.dot(xj, cf[...], preferred_element_type=f32) * wf


def _full(shape):
    return pl.BlockSpec(shape, lambda i: (0,) * len(shape))


def _edge_call(t0, g, eaux, weights):
    nblk = E_PAD // BE
    row = pl.BlockSpec((BE, H), lambda i: (i, 0))
    xj_spec = pl.BlockSpec((BE, H), lambda i: (i, 0))
    xi_spec = pl.BlockSpec((BE, H), lambda i: (i + nblk, 0))
    ein = NGP if t0 else H
    w_specs = [
        _full((H, 2 * H)), _full((H, 2 * H)), _full((ein, 2 * H)),
        _full((1, 2 * H)), _full((2 * H, H)), _full((1, H)),
        _full((H, H)), _full((1, H)), _full((H, H)), _full((1, H)),
        _full((H, H)),
    ]
    if t0:
        pos_spec = pl.BlockSpec((BE, 16), lambda i: (i, 0))
        pos_spec2 = pl.BlockSpec((BE, 16), lambda i: (i + nblk, 0))
        in_specs = [xj_spec, xi_spec, pos_spec, pos_spec2] + w_specs
        operands = (g, g, eaux, eaux) + weights
    else:
        ea_spec = pl.BlockSpec((BE, H), lambda i: (i, 0))
        in_specs = [xj_spec, xi_spec, ea_spec] + w_specs
        operands = (g, g, eaux) + weights
    out_shape = [jax.ShapeDtypeStruct((E_PAD, H), jnp.float32),
                 jax.ShapeDtypeStruct((E_PAD, H), jnp.float32)]
    body = lambda *refs: _edge_body(t0, refs)
    return pl.pallas_call(
        body,
        grid=(nblk,),
        in_specs=in_specs,
        out_specs=[row, row],
        out_shape=out_shape,
    )(*operands)


def _node_body(p_ref, h_ref, s1, b1, s2, b2, out_ref):
    msg = p_ref[0] + p_ref[1]
    u = _ssp(jnp.dot(msg, s1[...], preferred_element_type=jnp.float32) + b1[...])
    out_ref[...] = (h_ref[...] + jnp.dot(u, s2[...], preferred_element_type=jnp.float32)
                    + b2[...])


def _node_call(parts, h, s1, b1, s2, b2):
    nblk = N // BN
    return pl.pallas_call(
        _node_body,
        grid=(nblk,),
        in_specs=[
            pl.BlockSpec((2, BN, H), lambda i: (0, i, 0)),
            pl.BlockSpec((BN, H), lambda i: (i, 0)),
            _full((H, H)), _full((1, H)), _full((H, H)), _full((1, H)),
        ],
        out_specs=pl.BlockSpec((BN, H), lambda i: (i, 0)),
        out_shape=jax.ShapeDtypeStruct((N, H), jnp.float32),
    )(parts, h, s1, b1, s2, b2)


def _embed_body(z_ref, emb_ref, out_ref):
    ids = lax.broadcasted_iota(jnp.int32, (1, H), 1)
    oh = (z_ref[...] == ids).astype(jnp.float32)
    out_ref[...] = jnp.dot(oh, emb_ref[...], preferred_element_type=jnp.float32)


def _embed_call(z2d, emb_pad):
    nblk = N // BN
    return pl.pallas_call(
        _embed_body,
        grid=(nblk,),
        in_specs=[pl.BlockSpec((BN, 1), lambda i: (i, 0)), _full((H, H))],
        out_specs=pl.BlockSpec((BN, H), lambda i: (i, 0)),
        out_shape=jax.ShapeDtypeStruct((N, H), jnp.float32),
    )(z2d, emb_pad)


def _readout_body(h_ref, b_ref, l1, b1v, l2, b2v, out_ref):
    i = pl.program_id(0)
    t = _ssp(jnp.dot(h_ref[...], l1[...], preferred_element_type=jnp.float32) + b1v[...])
    node_out = jnp.dot(t, l2[...], preferred_element_type=jnp.float32) + b2v[...]
    ids = lax.broadcasted_iota(jnp.int32, (1, NGRAPHS), 1)
    oh = (b_ref[...] == ids).astype(jnp.float32)
    part = lax.dot_general(oh, node_out, (((0,), (0,)), ((), ())),
                           preferred_element_type=jnp.float32)

    @pl.when(i == 0)
    def _():
        out_ref[...] = part

    @pl.when(i > 0)
    def _():
        out_ref[...] = out_ref[...] + part


def _readout_call(h, b2d, l1, b1v, l2, b2v):
    nblk = N // BR
    return pl.pallas_call(
        _readout_body,
        grid=(nblk,),
        in_specs=[
            pl.BlockSpec((BR, H), lambda i: (i, 0)),
            pl.BlockSpec((BR, 1), lambda i: (i, 0)),
            _full((H, H // 2)), _full((1, H // 2)),
            _full((H // 2, 1)), _full((1, 1)),
        ],
        out_specs=pl.BlockSpec((NGRAPHS, 1), lambda i: (0, 0)),
        out_shape=jax.ShapeDtypeStruct((NGRAPHS, 1), jnp.float32),
    )(h, b2d, l1, b1v, l2, b2v)


def kernel(z, pos, edge_index, batch, params):
    src = edge_index[0].astype(jnp.int32)
    dst = edge_index[1].astype(jnp.int32)
    pad = E_PAD - E
    zpad = jnp.zeros((pad,), jnp.int32)
    srcp = jnp.concatenate([src, zpad])
    dstp = jnp.concatenate([dst, zpad])
    idx_sd = jnp.concatenate([srcp, dstp]).reshape(2 * E_PAD // CH, CH)
    dst_sc = jnp.concatenate([dst, jnp.full((pad,), N, jnp.int32)]
                             ).reshape(E_PAD // CH, CH)

    pos16 = jnp.zeros((N, 16), jnp.float32).at[:, :3].set(pos.astype(jnp.float32))
    emb_pad = jnp.zeros((H, H), jnp.float32).at[:NEMB].set(params['embedding'])
    zeros_ns = jnp.zeros((NS, H), jnp.float32)

    ppack = _gather_pos(pos16, idx_sd)
    h = _embed_call(z.astype(jnp.int32).reshape(N, 1), emb_pad)

    eattr = None
    for t, p in enumerate(params['interactions']):
        w1T = p['eu1_W'].T
        w1i = w1T[:H]
        w1j = w1T[H:2 * H]
        if t == 0:
            w1e = jnp.zeros((NGP, 2 * H), jnp.float32).at[:NGAUSS].set(w1T[2 * H:])
        else:
            w1e = w1T[2 * H:]
        weights = (
            w1i, w1j, w1e, p['eu1_b'].reshape(1, -1),
            p['eu2_W'].T, p['eu2_b'].reshape(1, -1),
            p['f1_W'].T, p['f1_b'].reshape(1, -1),
            p['f2_W'].T, p['f2_b'].reshape(1, -1),
            p['cf_W'].T,
        )
        g = _gather_h(h, idx_sd)
        eaux = ppack if t == 0 else eattr
        eattr, msg_e = _edge_call(t == 0, g, eaux, weights)
        parts = _scatter_msg(msg_e, dst_sc, zeros_ns)
        h = _node_call(parts, h,
                       p['sm1_W'].T, p['sm1_b'].reshape(1, -1),
                       p['sm2_W'].T, p['sm2_b'].reshape(1, -1))

    out = _readout_call(h, batch.astype(jnp.int32).reshape(N, 1),
                        params['lin1_W'].T, params['lin1_b'].reshape(1, -1),
                        params['lin2_W'].T, params['lin2_b'].reshape(1, 1))
    return out
